# Initial kernel scaffold; baseline (speedup 1.0000x reference)
import jax
import jax.numpy as jnp
from jax.experimental import pallas as pl


def kernel(x, embedding_matrix, W_ih, W_hh, b_ih, b_hh):
    raise NotImplementedError("write your pallas kernel here")


# speedup vs baseline: 3.2485x; 3.2485x over previous
import functools

import jax
import jax.numpy as jnp
from jax import lax
from jax.experimental import pallas as pl
from jax.experimental.pallas import tpu as pltpu
from jax.experimental.pallas import tpu_sc as plsc

VOCAB = 100000
EMB = 128
HID = 256
B = 1024
T = 50

_NC = 2
_NS = 16
_NW = _NC * _NS
_N_IDX = B * T
_PER_W = _N_IDX // _NW
_CHUNK = 80
_N_CHUNK = _PER_W // _CHUNK


def _sc_gather(table, idx):
    mesh = plsc.VectorSubcoreMesh(core_axis_name="c", subcore_axis_name="s")

    @functools.partial(
        pl.kernel,
        out_type=jax.ShapeDtypeStruct((_N_IDX, EMB), jnp.float32),
        mesh=mesh,
        scratch_types=[
            pltpu.VMEM((_CHUNK,), jnp.int32),
            pltpu.VMEM((_CHUNK, EMB), jnp.float32),
            pltpu.SemaphoreType.DMA,
        ],
    )
    def gather_kernel(table_hbm, idx_hbm, out_hbm, idx_v, rows_v, sem):
        wid = lax.axis_index("s") * _NC + lax.axis_index("c")
        base = wid * _PER_W

        def body(j, _):
            off = base + j * _CHUNK
            pltpu.sync_copy(idx_hbm.at[pl.ds(off, _CHUNK)], idx_v)
            pltpu.async_copy(table_hbm.at[idx_v], rows_v, sem).wait()
            pltpu.sync_copy(rows_v, out_hbm.at[pl.ds(off, _CHUNK)])
            return ()

        lax.fori_loop(0, _N_CHUNK, body, (), unroll=False)

    return gather_kernel(table, idx)


def _lstm_step(emb_ref, wih_ref, whh_ref, b_ref, h_out, c_out, h_s, c_s):
    t = pl.program_id(0)

    @pl.when(t == 0)
    def _():
        h_s[...] = jnp.zeros_like(h_s)
        c_s[...] = jnp.zeros_like(c_s)

    x = emb_ref[0]
    h = h_s[...]
    gates = (
        jnp.dot(x, wih_ref[...], preferred_element_type=jnp.float32)
        + jnp.dot(h, whh_ref[...], preferred_element_type=jnp.float32)
        + b_ref[...]
    )
    i = jax.nn.sigmoid(gates[:, 0 * HID : 1 * HID])
    f = jax.nn.sigmoid(gates[:, 1 * HID : 2 * HID])
    g = jnp.tanh(gates[:, 2 * HID : 3 * HID])
    o = jax.nn.sigmoid(gates[:, 3 * HID : 4 * HID])
    c_new = f * c_s[...] + i * g
    h_new = o * jnp.tanh(c_new)
    c_s[...] = c_new
    h_s[...] = h_new

    @pl.when(t == T - 1)
    def _():
        h_out[...] = h_new
        c_out[...] = c_new


def _tc_lstm(emb, wih_t, whh_t, bias):
    out_shape = [
        jax.ShapeDtypeStruct((B, HID), jnp.float32),
        jax.ShapeDtypeStruct((B, HID), jnp.float32),
    ]
    grid = (T,)
    return pl.pallas_call(
        _lstm_step,
        grid=grid,
        in_specs=[
            pl.BlockSpec((1, B, EMB), lambda t: (t, 0, 0)),
            pl.BlockSpec((EMB, 4 * HID), lambda t: (0, 0)),
            pl.BlockSpec((HID, 4 * HID), lambda t: (0, 0)),
            pl.BlockSpec((1, 4 * HID), lambda t: (0, 0)),
        ],
        out_specs=[
            pl.BlockSpec((B, HID), lambda t: (0, 0)),
            pl.BlockSpec((B, HID), lambda t: (0, 0)),
        ],
        out_shape=out_shape,
        scratch_shapes=[
            pltpu.VMEM((B, HID), jnp.float32),
            pltpu.VMEM((B, HID), jnp.float32),
        ],
    )(emb, wih_t, whh_t, bias)


def kernel(x, embedding_matrix, W_ih, W_hh, b_ih, b_hh):
    idx = jnp.reshape(jnp.transpose(x).astype(jnp.int32), (_N_IDX,))
    emb_flat = _sc_gather(embedding_matrix, idx)
    emb = jnp.reshape(emb_flat, (T, B, EMB))
    wih_t = jnp.transpose(W_ih)
    whh_t = jnp.transpose(W_hh)
    bias = jnp.reshape(b_ih + b_hh, (1, 4 * HID))
    h, c = _tc_lstm(emb, wih_t, whh_t, bias)
    return (h[None, :, :], c[None, :, :])


# speedup vs baseline: 3.4826x; 1.0721x over previous
import functools

import jax
import jax.numpy as jnp
from jax import lax
from jax.experimental import pallas as pl
from jax.experimental.pallas import tpu as pltpu
from jax.experimental.pallas import tpu_sc as plsc

VOCAB = 100000
EMB = 128
HID = 256
B = 1024
T = 50

_NC = 2
_NS = 16
_NW = _NC * _NS
_N_IDX = B * T
_PER_W = _N_IDX // _NW
_CHUNK = 80
_N_CHUNK = _PER_W // _CHUNK


def _sc_gather(table, idx):
    mesh = plsc.VectorSubcoreMesh(core_axis_name="c", subcore_axis_name="s")

    @functools.partial(
        pl.kernel,
        out_type=jax.ShapeDtypeStruct((_N_IDX, EMB), jnp.float32),
        mesh=mesh,
        scratch_types=[
            pltpu.VMEM((_CHUNK,), jnp.int32),
            pltpu.VMEM((_CHUNK, EMB), jnp.float32),
            pltpu.SemaphoreType.DMA,
        ],
    )
    def gather_kernel(table_hbm, idx_hbm, out_hbm, idx_v, rows_v, sem):
        wid = lax.axis_index("s") * _NC + lax.axis_index("c")
        base = wid * _PER_W

        def body(j, _):
            off = base + j * _CHUNK
            pltpu.sync_copy(idx_hbm.at[pl.ds(off, _CHUNK)], idx_v)
            pltpu.async_copy(table_hbm.at[idx_v], rows_v, sem).wait()
            pltpu.sync_copy(rows_v, out_hbm.at[pl.ds(off, _CHUNK)])
            return ()

        lax.fori_loop(0, _N_CHUNK, body, (), unroll=False)

    return gather_kernel(table, idx)


def _lstm_step(emb_ref, wih_ref, whh_ref, b_ref, h_out, c_out, h_s, c_s):
    t = pl.program_id(0)

    @pl.when(t == 0)
    def _():
        h_s[...] = jnp.zeros_like(h_s)
        c_s[...] = jnp.zeros_like(c_s)

    x = emb_ref[0]
    h = h_s[...]
    gates = (
        jnp.dot(x, wih_ref[...], preferred_element_type=jnp.float32)
        + jnp.dot(h, whh_ref[...], preferred_element_type=jnp.float32)
        + b_ref[...]
    )
    def _sig(v):
        return 0.5 * jnp.tanh(0.5 * v) + 0.5

    i = _sig(gates[:, 0 * HID : 1 * HID])
    f = _sig(gates[:, 1 * HID : 2 * HID])
    g = jnp.tanh(gates[:, 2 * HID : 3 * HID])
    o = _sig(gates[:, 3 * HID : 4 * HID])
    c_new = f * c_s[...] + i * g
    h_new = o * jnp.tanh(c_new)
    c_s[...] = c_new
    h_s[...] = h_new

    @pl.when(t == T - 1)
    def _():
        h_out[...] = h_new
        c_out[...] = c_new


def _tc_lstm(emb, wih_t, whh_t, bias):
    out_shape = [
        jax.ShapeDtypeStruct((B, HID), jnp.float32),
        jax.ShapeDtypeStruct((B, HID), jnp.float32),
    ]
    grid = (T,)
    return pl.pallas_call(
        _lstm_step,
        grid=grid,
        in_specs=[
            pl.BlockSpec((1, B, EMB), lambda t: (t, 0, 0)),
            pl.BlockSpec((EMB, 4 * HID), lambda t: (0, 0)),
            pl.BlockSpec((HID, 4 * HID), lambda t: (0, 0)),
            pl.BlockSpec((1, 4 * HID), lambda t: (0, 0)),
        ],
        out_specs=[
            pl.BlockSpec((B, HID), lambda t: (0, 0)),
            pl.BlockSpec((B, HID), lambda t: (0, 0)),
        ],
        out_shape=out_shape,
        scratch_shapes=[
            pltpu.VMEM((B, HID), jnp.float32),
            pltpu.VMEM((B, HID), jnp.float32),
        ],
    )(emb, wih_t, whh_t, bias)


def kernel(x, embedding_matrix, W_ih, W_hh, b_ih, b_hh):
    idx = jnp.reshape(jnp.transpose(x).astype(jnp.int32), (_N_IDX,))
    emb_flat = _sc_gather(embedding_matrix, idx)
    emb = jnp.reshape(emb_flat, (T, B, EMB))
    wih_t = jnp.transpose(W_ih)
    whh_t = jnp.transpose(W_hh)
    bias = jnp.reshape(b_ih + b_hh, (1, 4 * HID))
    h, c = _tc_lstm(emb, wih_t, whh_t, bias)
    return (h[None, :, :], c[None, :, :])


# speedup vs baseline: 3.5087x; 1.0075x over previous
import functools

import jax
import jax.numpy as jnp
from jax import lax
from jax.experimental import pallas as pl
from jax.experimental.pallas import tpu as pltpu
from jax.experimental.pallas import tpu_sc as plsc

VOCAB = 100000
EMB = 128
HID = 256
B = 1024
T = 50

_NC = 2
_NS = 16
_NW = _NC * _NS
_N_IDX = B * T
_PER_W = _N_IDX // _NW
_CHUNK = 80
_N_CHUNK = _PER_W // _CHUNK


def _sc_gather(table, idx):
    mesh = plsc.VectorSubcoreMesh(core_axis_name="c", subcore_axis_name="s")

    @functools.partial(
        pl.kernel,
        out_type=jax.ShapeDtypeStruct((_N_IDX, EMB), jnp.float32),
        mesh=mesh,
        scratch_types=[
            pltpu.VMEM((_CHUNK,), jnp.int32),
            pltpu.VMEM((_CHUNK, EMB), jnp.float32),
            pltpu.SemaphoreType.DMA,
        ],
    )
    def gather_kernel(table_hbm, idx_hbm, out_hbm, idx_v, rows_v, sem):
        wid = lax.axis_index("s") * _NC + lax.axis_index("c")
        base = wid * _PER_W

        def body(j, _):
            off = base + j * _CHUNK
            pltpu.sync_copy(idx_hbm.at[pl.ds(off, _CHUNK)], idx_v)
            pltpu.async_copy(table_hbm.at[idx_v], rows_v, sem).wait()
            pltpu.sync_copy(rows_v, out_hbm.at[pl.ds(off, _CHUNK)])
            return ()

        lax.fori_loop(0, _N_CHUNK, body, (), unroll=False)

    return gather_kernel(table, idx)


def _lstm_step(emb_ref, wih_ref, whh_ref, b_ref, h_out, c_out, h_s, c_s):
    t = pl.program_id(0)

    @pl.when(t == 0)
    def _():
        h_s[...] = jnp.zeros_like(h_s)
        c_s[...] = jnp.zeros_like(c_s)

    x = emb_ref[0].astype(jnp.bfloat16)
    h = h_s[...].astype(jnp.bfloat16)
    gates = (
        jnp.dot(x, wih_ref[...], preferred_element_type=jnp.float32)
        + jnp.dot(h, whh_ref[...], preferred_element_type=jnp.float32)
        + b_ref[...]
    )
    def _sig(v):
        return 0.5 * jnp.tanh(0.5 * v) + 0.5

    i = _sig(gates[:, 0 * HID : 1 * HID])
    f = _sig(gates[:, 1 * HID : 2 * HID])
    g = jnp.tanh(gates[:, 2 * HID : 3 * HID])
    o = _sig(gates[:, 3 * HID : 4 * HID])
    c_new = f * c_s[...] + i * g
    h_new = o * jnp.tanh(c_new)
    c_s[...] = c_new
    h_s[...] = h_new

    @pl.when(t == T - 1)
    def _():
        h_out[...] = h_new
        c_out[...] = c_new


def _tc_lstm(emb, wih_t, whh_t, bias):
    out_shape = [
        jax.ShapeDtypeStruct((B, HID), jnp.float32),
        jax.ShapeDtypeStruct((B, HID), jnp.float32),
    ]
    grid = (T,)
    return pl.pallas_call(
        _lstm_step,
        grid=grid,
        in_specs=[
            pl.BlockSpec((1, B, EMB), lambda t: (t, 0, 0)),
            pl.BlockSpec((EMB, 4 * HID), lambda t: (0, 0)),
            pl.BlockSpec((HID, 4 * HID), lambda t: (0, 0)),
            pl.BlockSpec((1, 4 * HID), lambda t: (0, 0)),
        ],
        out_specs=[
            pl.BlockSpec((B, HID), lambda t: (0, 0)),
            pl.BlockSpec((B, HID), lambda t: (0, 0)),
        ],
        out_shape=out_shape,
        scratch_shapes=[
            pltpu.VMEM((B, HID), jnp.float32),
            pltpu.VMEM((B, HID), jnp.float32),
        ],
    )(emb, wih_t, whh_t, bias)


def kernel(x, embedding_matrix, W_ih, W_hh, b_ih, b_hh):
    idx = jnp.reshape(jnp.transpose(x).astype(jnp.int32), (_N_IDX,))
    emb_flat = _sc_gather(embedding_matrix, idx)
    emb = jnp.reshape(emb_flat, (T, B, EMB))
    wih_t = jnp.transpose(W_ih).astype(jnp.bfloat16)
    whh_t = jnp.transpose(W_hh).astype(jnp.bfloat16)
    bias = jnp.reshape(b_ih + b_hh, (1, 4 * HID))
    h, c = _tc_lstm(emb, wih_t, whh_t, bias)
    return (h[None, :, :], c[None, :, :])


# speedup vs baseline: 4.1152x; 1.1729x over previous
import functools

import jax
import jax.numpy as jnp
from jax import lax
from jax.experimental import pallas as pl
from jax.experimental.pallas import tpu as pltpu
from jax.experimental.pallas import tpu_sc as plsc

VOCAB = 100000
EMB = 128
HID = 256
B = 1024
T = 50

_NC = 2
_NS = 16
_NW = _NC * _NS
_N_IDX = B * T
_PER_W = _N_IDX // _NW
_CHUNK = 80
_N_CHUNK = _PER_W // _CHUNK


_NBUF = 4


def _sc_gather(table, idx2d):
    mesh = plsc.VectorSubcoreMesh(core_axis_name="c", subcore_axis_name="s")

    @functools.partial(
        pl.kernel,
        out_type=jax.ShapeDtypeStruct((_N_IDX, EMB), jnp.float32),
        mesh=mesh,
        scratch_types=[
            pltpu.VMEM((_N_CHUNK, _CHUNK), jnp.int32),
            [pltpu.VMEM((_CHUNK, EMB), jnp.float32) for _ in range(_NBUF)],
            [pltpu.SemaphoreType.DMA for _ in range(_NBUF)],
            [pltpu.SemaphoreType.DMA for _ in range(_NBUF)],
        ],
    )
    def gather_kernel(table_hbm, idx_hbm, out_hbm, idx_v, bufs, gsems, ssems):
        wid = lax.axis_index("s") * _NC + lax.axis_index("c")
        base = wid * _PER_W
        pltpu.sync_copy(idx_hbm.at[wid], idx_v)

        gathers = [None] * _N_CHUNK
        stores = [None] * _N_CHUNK

        def start_gather(j):
            b = j % _NBUF
            gathers[j] = pltpu.async_copy(
                table_hbm.at[idx_v.at[j]], bufs[b], gsems[b]
            )

        for j in range(_NBUF):
            start_gather(j)
        for j in range(_N_CHUNK):
            b = j % _NBUF
            gathers[j].wait()
            stores[j] = pltpu.async_copy(
                bufs[b], out_hbm.at[pl.ds(base + j * _CHUNK, _CHUNK)], ssems[b]
            )
            nxt = j + _NBUF
            if nxt < _N_CHUNK:
                stores[j].wait()
                start_gather(nxt)
        for j in range(_N_CHUNK - _NBUF, _N_CHUNK):
            stores[j].wait()

    return gather_kernel(table, idx2d)


def _lstm_step(emb_ref, wih_ref, whh_ref, b_ref, h_out, c_out, h_s, c_s):
    t = pl.program_id(0)

    @pl.when(t == 0)
    def _():
        h_s[...] = jnp.zeros_like(h_s)
        c_s[...] = jnp.zeros_like(c_s)

    x = emb_ref[0].astype(jnp.bfloat16)
    h = h_s[...].astype(jnp.bfloat16)
    gates = (
        jnp.dot(x, wih_ref[...], preferred_element_type=jnp.float32)
        + jnp.dot(h, whh_ref[...], preferred_element_type=jnp.float32)
        + b_ref[...]
    )
    def _sig(v):
        return 0.5 * jnp.tanh(0.5 * v) + 0.5

    i = _sig(gates[:, 0 * HID : 1 * HID])
    f = _sig(gates[:, 1 * HID : 2 * HID])
    g = jnp.tanh(gates[:, 2 * HID : 3 * HID])
    o = _sig(gates[:, 3 * HID : 4 * HID])
    c_new = f * c_s[...] + i * g
    h_new = o * jnp.tanh(c_new)
    c_s[...] = c_new
    h_s[...] = h_new

    @pl.when(t == T - 1)
    def _():
        h_out[...] = h_new
        c_out[...] = c_new


def _tc_lstm(emb, wih_t, whh_t, bias):
    out_shape = [
        jax.ShapeDtypeStruct((B, HID), jnp.float32),
        jax.ShapeDtypeStruct((B, HID), jnp.float32),
    ]
    grid = (T,)
    return pl.pallas_call(
        _lstm_step,
        grid=grid,
        in_specs=[
            pl.BlockSpec((1, B, EMB), lambda t: (t, 0, 0)),
            pl.BlockSpec((EMB, 4 * HID), lambda t: (0, 0)),
            pl.BlockSpec((HID, 4 * HID), lambda t: (0, 0)),
            pl.BlockSpec((1, 4 * HID), lambda t: (0, 0)),
        ],
        out_specs=[
            pl.BlockSpec((B, HID), lambda t: (0, 0)),
            pl.BlockSpec((B, HID), lambda t: (0, 0)),
        ],
        out_shape=out_shape,
        scratch_shapes=[
            pltpu.VMEM((B, HID), jnp.float32),
            pltpu.VMEM((B, HID), jnp.float32),
        ],
    )(emb, wih_t, whh_t, bias)


def kernel(x, embedding_matrix, W_ih, W_hh, b_ih, b_hh):
    idx3d = jnp.reshape(
        jnp.transpose(x).astype(jnp.int32), (_NW, _N_CHUNK, _CHUNK)
    )
    emb_flat = _sc_gather(embedding_matrix, idx3d)
    emb = jnp.reshape(emb_flat, (T, B, EMB))
    wih_t = jnp.transpose(W_ih).astype(jnp.bfloat16)
    whh_t = jnp.transpose(W_hh).astype(jnp.bfloat16)
    bias = jnp.reshape(b_ih + b_hh, (1, 4 * HID))
    h, c = _tc_lstm(emb, wih_t, whh_t, bias)
    return (h[None, :, :], c[None, :, :])
